# baseline (speedup 1.0000x reference)
import functools

import jax
import jax.numpy as jnp
from jax import lax
from jax.experimental import pallas as pl
from jax.experimental.pallas import tpu as pltpu
from jax.experimental.pallas import tpu_sc as plsc

_N_CHANNELS = 384
_J = 128
_N_SPIKES = 2_000_000
_C = 96
_MIN_COVERAGE = 0.25

_NC, _NS, _L = 2, 16, 16
_NW = _NC * _NS

_VREGS = _N_SPIKES // (_NW * _L)
_WORDS_PER = _VREGS * _L
_TAIL_WORDS = _N_SPIKES - _NW * _WORDS_PER
_NCHUNK = 9
_CH_VREGS = _VREGS // _NCHUNK
_CH_WORDS = _CH_VREGS * _L
_UNROLL = 7
_CH_ITERS = _CH_VREGS // _UNROLL
_HIST = _L * _J

assert _CH_VREGS * _NCHUNK == _VREGS and _CH_ITERS * _UNROLL == _CH_VREGS


def _sc_hist(ids_lo):
    mesh = plsc.VectorSubcoreMesh(core_axis_name="c", subcore_axis_name="s")

    @functools.partial(
        pl.kernel,
        out_type=jax.ShapeDtypeStruct((_NW, _HIST), jnp.int32),
        mesh=mesh,
        scratch_types=[
            pltpu.VMEM((_CH_WORDS,), jnp.uint32),
            pltpu.VMEM((_CH_WORDS,), jnp.uint32),
            pltpu.VMEM((_HIST,), jnp.int32),
            pltpu.VMEM((_TAIL_WORDS,), jnp.uint32),
            pltpu.SemaphoreType.DMA,
            pltpu.SemaphoreType.DMA,
        ],
        compiler_params=pltpu.CompilerParams(needs_layout_passes=False),
    )
    def k(ids_hbm, out_hbm, buf0, buf1, hist, tbuf, sem0, sem1):
        i32 = jnp.int32
        wid = (lax.axis_index("s") * i32(_NC) + lax.axis_index("c")).astype(i32)
        lane = lax.iota(i32, _L)
        ones = jnp.full((_L,), 1, i32)
        zeros = jnp.zeros((_L,), i32)
        base = wid * i32(_WORDS_PER)

        bufs = (buf0, buf1)
        sems = (sem0, sem1)

        def zbody(i, c):
            s = i * i32(_L * 8)
            for u in range(8):
                hist[pl.ds(s + i32(u * _L), _L)] = zeros
            return c

        lax.fori_loop(i32(0), i32(_HIST // (_L * 8)), zbody, i32(0))

        copies = [None] * _NCHUNK
        copies[0] = pltpu.async_copy(
            ids_hbm.at[pl.ds(base, _CH_WORDS)], buf0, sem0)
        for j in range(_NCHUNK):
            if j + 1 < _NCHUNK:
                copies[j + 1] = pltpu.async_copy(
                    ids_hbm.at[pl.ds(base + i32((j + 1) * _CH_WORDS),
                                     _CH_WORDS)],
                    bufs[(j + 1) % 2], sems[(j + 1) % 2])
            copies[j].wait()
            b = bufs[j % 2]

            @plsc.parallel_loop(i32(0), i32(_CH_VREGS), i32(1),
                                unroll=_UNROLL)
            def _(i, b=b):
                v = plsc.bitcast(b[pl.ds(i * i32(_L), _L)], jnp.int32)
                plsc.addupdate_scatter(hist, [(v << 4) + lane], ones)

        @pl.when(wid == 0)
        def _():
            pltpu.sync_copy(ids_hbm.at[pl.ds(_NW * _WORDS_PER, _TAIL_WORDS)],
                            tbuf)

            def tbody(i, c):
                v = plsc.bitcast(tbuf[pl.ds(i * i32(_L), _L)], jnp.int32)
                plsc.addupdate_scatter(hist, [(v << 4) + lane], ones)
                return c

            lax.fori_loop(i32(0), i32(_TAIL_WORDS // _L), tbody, i32(0))

        pltpu.sync_copy(hist, out_hbm.at[wid])

    return k(ids_lo)


def _tc_cov(channels_2d, indicators):
    def body(ch_ref, ind_ref, num_ref, cnt_ref):
        ioc = lax.broadcasted_iota(jnp.int32, (_C, _N_CHANNELS), 1)
        eq = (ch_ref[...] == ioc).astype(jnp.float32)
        w = jnp.sum(eq, axis=0, keepdims=True)
        num_ref[...] = jnp.dot(w, ind_ref[...],
                               precision=lax.Precision.HIGHEST,
                               preferred_element_type=jnp.float32)
        cnt_ref[...] = jnp.sum(ind_ref[...], axis=0, keepdims=True)

    return pl.pallas_call(
        body,
        out_shape=(
            jax.ShapeDtypeStruct((1, _J), jnp.float32),
            jax.ShapeDtypeStruct((1, _J), jnp.float32),
        ),
    )(channels_2d, indicators)


def _tc_finish(partials, num, counts):
    def body(parts_ref, num_ref, cnt_ref, cov_ref, covf_ref, nsp_ref):
        p = parts_ref[...].astype(jnp.float32)
        psum = jnp.sum(p, axis=0, keepdims=True)
        row = lax.broadcasted_iota(jnp.int32, (_HIST, _J), 0)
        col = lax.broadcasted_iota(jnp.int32, (_HIST, _J), 1)
        onehot = ((row >> 4) == col).astype(jnp.float32)
        pops_f = jnp.dot(psum, onehot,
                         precision=lax.Precision.HIGHEST,
                         preferred_element_type=jnp.float32)
        coverage = num_ref[...] / cnt_ref[...]
        covered = coverage >= _MIN_COVERAGE
        cov_ref[...] = coverage
        covf_ref[...] = covered.astype(jnp.float32)
        nsp_f = jnp.sum(jnp.where(covered, pops_f, jnp.zeros_like(pops_f)))
        nsp_ref[0, 0] = nsp_f.astype(jnp.int32)

    return pl.pallas_call(
        body,
        out_shape=(
            jax.ShapeDtypeStruct((1, _J), jnp.float32),
            jax.ShapeDtypeStruct((1, _J), jnp.float32),
            jax.ShapeDtypeStruct((1, 1), jnp.int32),
        ),
        out_specs=(
            pl.BlockSpec(memory_space=pltpu.VMEM),
            pl.BlockSpec(memory_space=pltpu.VMEM),
            pl.BlockSpec(memory_space=pltpu.SMEM),
        ),
    )(partials, num, counts)


def kernel(neighborhood_ids, neighborhoods, channels, indicators):
    ids_lo = neighborhood_ids.astype(jnp.uint32)
    ch2d = channels.astype(jnp.int32).reshape(_C, 1)
    partials = _sc_hist(ids_lo)
    num, counts = _tc_cov(ch2d, indicators)
    cov, covf, nsp = _tc_finish(partials, num, counts)
    return (cov.reshape(_J), covf.reshape(_J),
            nsp.reshape(()).astype(jnp.int64))


# speedup vs baseline: 1.0135x; 1.0135x over previous
import functools

import jax
import jax.numpy as jnp
from jax import lax
from jax.experimental import pallas as pl
from jax.experimental.pallas import tpu as pltpu
from jax.experimental.pallas import tpu_sc as plsc

_N_CHANNELS = 384
_J = 128
_N_SPIKES = 2_000_000
_C = 96
_MIN_COVERAGE = 0.25

_NC, _NS, _L = 2, 16, 16
_NW = _NC * _NS

_VREGS = _N_SPIKES // (_NW * _L)
_WORDS_PER = _VREGS * _L
_TAIL_WORDS = _N_SPIKES - _NW * _WORDS_PER
_NCHUNK = 6
_CH_VREGS = _VREGS // _NCHUNK
_CH_WORDS = _CH_VREGS * _L
_UNROLL = 7
_CH_ITERS = _CH_VREGS // _UNROLL
_HIST = _L * _J

assert _CH_VREGS * _NCHUNK == _VREGS and _CH_ITERS * _UNROLL == _CH_VREGS


def _sc_hist(ids_lo):
    mesh = plsc.VectorSubcoreMesh(core_axis_name="c", subcore_axis_name="s")

    @functools.partial(
        pl.kernel,
        out_type=jax.ShapeDtypeStruct((_NW, _HIST), jnp.int32),
        mesh=mesh,
        scratch_types=[
            pltpu.VMEM((_CH_WORDS,), jnp.uint32),
            pltpu.VMEM((_CH_WORDS,), jnp.uint32),
            pltpu.VMEM((_CH_WORDS,), jnp.uint32),
            pltpu.VMEM((_HIST,), jnp.int32),
            pltpu.VMEM((_TAIL_WORDS,), jnp.uint32),
            pltpu.SemaphoreType.DMA,
            pltpu.SemaphoreType.DMA,
            pltpu.SemaphoreType.DMA,
        ],
        compiler_params=pltpu.CompilerParams(needs_layout_passes=False),
    )
    def k(ids_hbm, out_hbm, buf0, buf1, buf2, hist, tbuf, sem0, sem1, sem2):
        i32 = jnp.int32
        wid = (lax.axis_index("s") * i32(_NC) + lax.axis_index("c")).astype(i32)
        lane = lax.iota(i32, _L)
        ones = jnp.full((_L,), 1, i32)
        zeros = jnp.zeros((_L,), i32)
        base = wid * i32(_WORDS_PER)

        bufs = (buf0, buf1, buf2)
        sems = (sem0, sem1, sem2)

        def zbody(i, c):
            s = i * i32(_L * 8)
            for u in range(8):
                hist[pl.ds(s + i32(u * _L), _L)] = zeros
            return c

        lax.fori_loop(i32(0), i32(_HIST // (_L * 8)), zbody, i32(0))

        copies = [None] * _NCHUNK
        for j in range(2):
            copies[j] = pltpu.async_copy(
                ids_hbm.at[pl.ds(base + i32(j * _CH_WORDS), _CH_WORDS)],
                bufs[j % 3], sems[j % 3])
        for j in range(_NCHUNK):
            if j + 2 < _NCHUNK:
                copies[j + 2] = pltpu.async_copy(
                    ids_hbm.at[pl.ds(base + i32((j + 2) * _CH_WORDS),
                                     _CH_WORDS)],
                    bufs[(j + 2) % 3], sems[(j + 2) % 3])
            copies[j].wait()
            b = bufs[j % 3]

            @plsc.parallel_loop(i32(0), i32(_CH_VREGS), i32(1),
                                unroll=_UNROLL)
            def _(i, b=b):
                v = plsc.bitcast(b[pl.ds(i * i32(_L), _L)], jnp.int32)
                plsc.addupdate_scatter(hist, [(v << 4) + lane], ones)

        @pl.when(wid == 0)
        def _():
            pltpu.sync_copy(ids_hbm.at[pl.ds(_NW * _WORDS_PER, _TAIL_WORDS)],
                            tbuf)

            def tbody(i, c):
                v = plsc.bitcast(tbuf[pl.ds(i * i32(_L), _L)], jnp.int32)
                plsc.addupdate_scatter(hist, [(v << 4) + lane], ones)
                return c

            lax.fori_loop(i32(0), i32(_TAIL_WORDS // _L), tbody, i32(0))

        pltpu.sync_copy(hist, out_hbm.at[wid])

    return k(ids_lo)


def _tc_cov(channels_2d, indicators):
    def body(ch_ref, ind_ref, num_ref, cnt_ref):
        ioc = lax.broadcasted_iota(jnp.int32, (_C, _N_CHANNELS), 1)
        eq = (ch_ref[...] == ioc).astype(jnp.float32)
        w = jnp.sum(eq, axis=0, keepdims=True)
        num_ref[...] = jnp.dot(w, ind_ref[...],
                               precision=lax.Precision.HIGHEST,
                               preferred_element_type=jnp.float32)
        cnt_ref[...] = jnp.sum(ind_ref[...], axis=0, keepdims=True)

    return pl.pallas_call(
        body,
        out_shape=(
            jax.ShapeDtypeStruct((1, _J), jnp.float32),
            jax.ShapeDtypeStruct((1, _J), jnp.float32),
        ),
    )(channels_2d, indicators)


def _tc_finish(partials, num, counts):
    def body(parts_ref, num_ref, cnt_ref, cov_ref, covf_ref, nsp_ref):
        p = parts_ref[...].astype(jnp.float32)
        psum = jnp.sum(p, axis=0, keepdims=True)
        row = lax.broadcasted_iota(jnp.int32, (_HIST, _J), 0)
        col = lax.broadcasted_iota(jnp.int32, (_HIST, _J), 1)
        onehot = ((row >> 4) == col).astype(jnp.float32)
        pops_f = jnp.dot(psum, onehot,
                         precision=lax.Precision.HIGHEST,
                         preferred_element_type=jnp.float32)
        coverage = num_ref[...] / cnt_ref[...]
        covered = coverage >= _MIN_COVERAGE
        cov_ref[...] = coverage
        covf_ref[...] = covered.astype(jnp.float32)
        nsp_f = jnp.sum(jnp.where(covered, pops_f, jnp.zeros_like(pops_f)))
        nsp_ref[0, 0] = nsp_f.astype(jnp.int32)

    return pl.pallas_call(
        body,
        out_shape=(
            jax.ShapeDtypeStruct((1, _J), jnp.float32),
            jax.ShapeDtypeStruct((1, _J), jnp.float32),
            jax.ShapeDtypeStruct((1, 1), jnp.int32),
        ),
        out_specs=(
            pl.BlockSpec(memory_space=pltpu.VMEM),
            pl.BlockSpec(memory_space=pltpu.VMEM),
            pl.BlockSpec(memory_space=pltpu.SMEM),
        ),
    )(partials, num, counts)


def kernel(neighborhood_ids, neighborhoods, channels, indicators):
    ids_lo = neighborhood_ids.astype(jnp.uint32)
    ch2d = channels.astype(jnp.int32).reshape(_C, 1)
    partials = _sc_hist(ids_lo)
    num, counts = _tc_cov(ch2d, indicators)
    cov, covf, nsp = _tc_finish(partials, num, counts)
    return (cov.reshape(_J), covf.reshape(_J),
            nsp.reshape(()).astype(jnp.int64))
